# Initial kernel scaffold; baseline (speedup 1.0000x reference)
import jax
import jax.numpy as jnp
from jax.experimental import pallas as pl


def kernel(x, edge_index, W0, b0, W1, b1, W2, b2, W3, b3, W4, b4, W5, b5, Wfc, bfc):
    raise NotImplementedError("write your pallas kernel here")


# speedup vs baseline: 19.0146x; 19.0146x over previous
import functools

import jax
import jax.numpy as jnp
from jax import lax
from jax.experimental import pallas as pl
from jax.experimental.pallas import tpu as pltpu
from jax.experimental.pallas import tpu_sc as plsc

N = 10000
E = 320000
D = 128
H = 16
C = 40

NC = 2
NS = 16
NW = NC * NS

BLK = 128
CHUNKS = -(-(-(-E // (NW * BLK))) // 8) * 8
EPAD = NW * BLK * CHUNKS
NROW = 10112
RPS = NROW // NS

_f32 = jnp.float32


def _sc_mesh():
    return plsc.VectorSubcoreMesh(core_axis_name="c", subcore_axis_name="s", )


def _deg_body(dst_hbm, out_hbm, dst_idx, ones_v, zbuf, acc):
    cid = lax.axis_index("c")
    sid = lax.axis_index("s")
    wid = sid * NC + cid
    ebase = pl.multiple_of(wid * CHUNKS, 8)
    rbase = pl.multiple_of(sid * RPS, 8)
    obase = pl.multiple_of(cid * NROW + sid * RPS, 8)

    def fill_ones(j, _):
        ones_v[j, :] = jnp.ones((H,), _f32)
        return 0

    def fill_zero(j, _):
        zbuf[j, :] = jnp.zeros((H,), _f32)
        return 0

    lax.fori_loop(0, BLK, fill_ones, 0)
    lax.fori_loop(0, RPS, fill_zero, 0)
    pltpu.sync_copy(zbuf, acc.at[pl.ds(rbase, RPS)])
    pltpu.sync_copy(dst_hbm.at[pl.ds(ebase, CHUNKS)], dst_idx)
    plsc.subcore_barrier()

    def step(i, _):
        pltpu.sync_copy(ones_v, acc.at[dst_idx.at[i]], add=True)
        return 0

    lax.fori_loop(0, CHUNKS, step, 0)
    plsc.subcore_barrier()
    pltpu.sync_copy(acc.at[pl.ds(rbase, RPS)], zbuf)
    pltpu.sync_copy(zbuf, out_hbm.at[pl.ds(obase, RPS)])


_deg_call = pl.kernel(
    _deg_body,
    out_type=jax.ShapeDtypeStruct((NC * NROW, H), _f32),
    mesh=_sc_mesh(),
    scratch_types=[
        pltpu.VMEM((CHUNKS, BLK), jnp.int32),
        pltpu.VMEM((BLK, H), _f32),
        pltpu.VMEM((RPS, H), _f32),
        pltpu.VMEM_SHARED((NROW, H), _f32),
    ],
    compiler_params=pltpu.CompilerParams(use_tc_tiling_on_sc=False),
)


def _prop_body(y_hbm, src_hbm, dst_hbm, out_hbm,
               src_idx, dst_idx, rows, zbuf, acc, sem):
    cid = lax.axis_index("c")
    sid = lax.axis_index("s")
    wid = sid * NC + cid
    ebase = pl.multiple_of(wid * CHUNKS, 8)
    rbase = pl.multiple_of(sid * RPS, 8)
    obase = pl.multiple_of(cid * NROW + sid * RPS, 8)

    def fill_zero(j, _):
        zbuf[j, :] = jnp.zeros((H,), _f32)
        return 0

    lax.fori_loop(0, RPS, fill_zero, 0)
    pltpu.sync_copy(zbuf, acc.at[pl.ds(rbase, RPS)])
    pltpu.sync_copy(src_hbm.at[pl.ds(ebase, CHUNKS)], src_idx)
    pltpu.sync_copy(dst_hbm.at[pl.ds(ebase, CHUNKS)], dst_idx)
    plsc.subcore_barrier()

    def step(i, _):
        pltpu.async_copy(y_hbm.at[src_idx.at[i]], rows, sem).wait()
        pltpu.sync_copy(rows, acc.at[dst_idx.at[i]], add=True)
        return 0

    lax.fori_loop(0, CHUNKS, step, 0)
    plsc.subcore_barrier()
    pltpu.sync_copy(acc.at[pl.ds(rbase, RPS)], zbuf)
    pltpu.sync_copy(zbuf, out_hbm.at[pl.ds(obase, RPS)])


_prop_call = pl.kernel(
    _prop_body,
    out_type=jax.ShapeDtypeStruct((NC * NROW, H), _f32),
    mesh=_sc_mesh(),
    scratch_types=[
        pltpu.VMEM((CHUNKS, BLK), jnp.int32),
        pltpu.VMEM((CHUNKS, BLK), jnp.int32),
        pltpu.VMEM((BLK, H), _f32),
        pltpu.VMEM((RPS, H), _f32),
        pltpu.VMEM_SHARED((NROW, H), _f32),
        pltpu.SemaphoreType.DMA,
    ],
    compiler_params=pltpu.CompilerParams(use_tc_tiling_on_sc=False),
)


def _pre_body(dd_ref, x_ref, w0_ref, dis_ref, y0_ref):
    indeg = dd_ref[0:N, :] + dd_ref[NROW:NROW + N, :]
    dis = lax.rsqrt(indeg + 1.0)
    z = jnp.dot(x_ref[...], w0_ref[...], preferred_element_type=_f32)
    dis_ref[...] = dis
    y0_ref[...] = dis * z


_pre_call = pl.pallas_call(
    _pre_body,
    out_shape=(jax.ShapeDtypeStruct((N, H), _f32),
               jax.ShapeDtypeStruct((N, H), _f32)),
)


def _layer_body(pp_ref, y_ref, dis_ref, m_ref, b_ref, w_ref,
                ynext_ref, mout_ref):
    S = pp_ref[0:N, :] + pp_ref[NROW:NROW + N, :]
    dis = dis_ref[...]
    h = jnp.maximum(dis * (S + y_ref[...]) + b_ref[...], 0.0)
    mout_ref[...] = jnp.maximum(m_ref[...], h)
    ynext_ref[...] = dis * jnp.dot(h, w_ref[...],
                                   preferred_element_type=_f32)


_layer_call = pl.pallas_call(
    _layer_body,
    out_shape=(jax.ShapeDtypeStruct((N, H), _f32),
               jax.ShapeDtypeStruct((N, H), _f32)),
)


def _final_body(pp_ref, y_ref, dis_ref, m_ref, b_ref, wfc_ref, bfc_ref,
                out_ref):
    S = pp_ref[0:N, :] + pp_ref[NROW:NROW + N, :]
    h = jnp.maximum(dis_ref[...] * (S + y_ref[...]) + b_ref[...], 0.0)
    m = jnp.maximum(m_ref[...], h)
    logits = jnp.dot(m, wfc_ref[...], preferred_element_type=_f32)
    logits = logits + bfc_ref[...]
    lmax = jnp.max(logits, axis=1, keepdims=True)
    s = logits - lmax
    out_ref[...] = s - jnp.log(jnp.sum(jnp.exp(s), axis=1, keepdims=True))


_final_call = pl.pallas_call(
    _final_body,
    out_shape=jax.ShapeDtypeStruct((N, C), _f32),
)


def kernel(x, edge_index, W0, b0, W1, b1, W2, b2, W3, b3, W4, b4, W5, b5,
           Wfc, bfc):
    src = edge_index[0].astype(jnp.int32)
    dst = edge_index[1].astype(jnp.int32)
    pad = EPAD - E
    srcr = jnp.concatenate([src, jnp.zeros((pad,), jnp.int32)])
    dstr = jnp.concatenate([dst, jnp.full((pad,), N, jnp.int32)])
    srcr = srcr.reshape(EPAD // BLK, BLK)
    dstr = dstr.reshape(EPAD // BLK, BLK)

    dd = _deg_call(dstr)
    dis, y = _pre_call(dd, x, W0)

    m = jnp.zeros((N, H), _f32)
    Ws = [W1, W2, W3, W4, W5]
    bs = [b0.reshape(1, H), b1.reshape(1, H), b2.reshape(1, H),
          b3.reshape(1, H), b4.reshape(1, H)]
    for l in range(5):
        pp = _prop_call(y, srcr, dstr)
        y, m = _layer_call(pp, y, dis, m, bs[l], Ws[l])
    pp = _prop_call(y, srcr, dstr)
    return _final_call(pp, y, dis, m, b5.reshape(1, H), Wfc,
                       bfc.reshape(1, C))


# speedup vs baseline: 26.3334x; 1.3849x over previous
import functools

import jax
import jax.numpy as jnp
from jax import lax
from jax.experimental import pallas as pl
from jax.experimental.pallas import tpu as pltpu
from jax.experimental.pallas import tpu_sc as plsc

N = 10000
E = 320000
D = 128
H = 16
C = 40

NC = 2
NS = 16
NW = NC * NS

BLK = 128
CHUNKS = -(-(-(-E // (NW * BLK))) // 8) * 8
EPAD = NW * BLK * CHUNKS
NROW = 10112
RPS = NROW // NS

_f32 = jnp.float32


def _sc_mesh():
    return plsc.VectorSubcoreMesh(core_axis_name="c", subcore_axis_name="s", )


def _deg_body(dst_hbm, out_hbm, dst_idx, ones_v, zbuf, acc):
    cid = lax.axis_index("c")
    sid = lax.axis_index("s")
    wid = sid * NC + cid
    ebase = pl.multiple_of(wid * CHUNKS, 8)
    rbase = pl.multiple_of(sid * RPS, 8)
    obase = pl.multiple_of(cid * NROW + sid * RPS, 8)

    def fill_ones(j, _):
        ones_v[j, :] = jnp.ones((H,), _f32)
        return 0

    def fill_zero(j, _):
        zbuf[j, :] = jnp.zeros((H,), _f32)
        return 0

    lax.fori_loop(0, BLK, fill_ones, 0)
    lax.fori_loop(0, RPS, fill_zero, 0)
    pltpu.sync_copy(zbuf, acc.at[pl.ds(rbase, RPS)])
    pltpu.sync_copy(dst_hbm.at[pl.ds(ebase, CHUNKS)], dst_idx)
    plsc.subcore_barrier()

    def step(i, _):
        pltpu.sync_copy(ones_v, acc.at[dst_idx.at[i]], add=True)
        return 0

    lax.fori_loop(0, CHUNKS, step, 0)
    plsc.subcore_barrier()
    pltpu.sync_copy(acc.at[pl.ds(rbase, RPS)], zbuf)
    pltpu.sync_copy(zbuf, out_hbm.at[pl.ds(obase, RPS)])


_deg_call = pl.kernel(
    _deg_body,
    out_type=jax.ShapeDtypeStruct((NC * NROW, H), _f32),
    mesh=_sc_mesh(),
    scratch_types=[
        pltpu.VMEM((CHUNKS, BLK), jnp.int32),
        pltpu.VMEM((BLK, H), _f32),
        pltpu.VMEM((RPS, H), _f32),
        pltpu.VMEM_SHARED((NROW, H), _f32),
    ],
    compiler_params=pltpu.CompilerParams(use_tc_tiling_on_sc=False),
)


NBUF = 4


def _prop_body(y_hbm, src_hbm, dst_hbm, out_hbm,
               src_idx, dst_idx, bufs, zbuf, acc, sems):
    cid = lax.axis_index("c")
    sid = lax.axis_index("s")
    wid = sid * NC + cid
    ebase = pl.multiple_of(wid * CHUNKS, 8)
    rbase = pl.multiple_of(sid * RPS, 8)
    obase = pl.multiple_of(cid * NROW + sid * RPS, 8)

    def fill_zero(j, _):
        zbuf[j, :] = jnp.zeros((H,), _f32)
        return 0

    lax.fori_loop(0, RPS, fill_zero, 0)
    pltpu.sync_copy(zbuf, acc.at[pl.ds(rbase, RPS)])
    pltpu.sync_copy(src_hbm.at[pl.ds(ebase, CHUNKS)], src_idx)
    pltpu.sync_copy(dst_hbm.at[pl.ds(ebase, CHUNKS)], dst_idx)
    plsc.subcore_barrier()

    for b in range(NBUF):
        pltpu.async_copy(y_hbm.at[src_idx.at[b]], bufs.at[b], sems.at[b])

    def group(g, _):
        for b in range(NBUF):
            i = g * NBUF + b
            pltpu.make_async_copy(y_hbm.at[src_idx.at[i]], bufs.at[b],
                                  sems.at[b]).wait()
            pltpu.sync_copy(bufs.at[b], acc.at[dst_idx.at[i]], add=True)

            @pl.when(i + NBUF < CHUNKS)
            def _():
                pltpu.async_copy(y_hbm.at[src_idx.at[i + NBUF]], bufs.at[b],
                                 sems.at[b])
        return 0

    lax.fori_loop(0, CHUNKS // NBUF, group, 0)
    plsc.subcore_barrier()
    pltpu.sync_copy(acc.at[pl.ds(rbase, RPS)], zbuf)
    pltpu.sync_copy(zbuf, out_hbm.at[pl.ds(obase, RPS)])


_prop_call = pl.kernel(
    _prop_body,
    out_type=jax.ShapeDtypeStruct((NC * NROW, H), _f32),
    mesh=_sc_mesh(),
    scratch_types=[
        pltpu.VMEM((CHUNKS, BLK), jnp.int32),
        pltpu.VMEM((CHUNKS, BLK), jnp.int32),
        pltpu.VMEM((NBUF, BLK, H), _f32),
        pltpu.VMEM((RPS, H), _f32),
        pltpu.VMEM_SHARED((NROW, H), _f32),
        pltpu.SemaphoreType.DMA((NBUF,)),
    ],
    compiler_params=pltpu.CompilerParams(use_tc_tiling_on_sc=False),
)


def _pre_body(dd_ref, x_ref, w0_ref, dis_ref, y0_ref):
    indeg = dd_ref[0:N, :] + dd_ref[NROW:NROW + N, :]
    dis = lax.rsqrt(indeg + 1.0)
    z = jnp.dot(x_ref[...], w0_ref[...], preferred_element_type=_f32)
    dis_ref[...] = dis
    y0_ref[...] = dis * z


_pre_call = pl.pallas_call(
    _pre_body,
    out_shape=(jax.ShapeDtypeStruct((N, H), _f32),
               jax.ShapeDtypeStruct((N, H), _f32)),
)


def _layer_body(pp_ref, y_ref, dis_ref, m_ref, b_ref, w_ref,
                ynext_ref, mout_ref):
    S = pp_ref[0:N, :] + pp_ref[NROW:NROW + N, :]
    dis = dis_ref[...]
    h = jnp.maximum(dis * (S + y_ref[...]) + b_ref[...], 0.0)
    mout_ref[...] = jnp.maximum(m_ref[...], h)
    ynext_ref[...] = dis * jnp.dot(h, w_ref[...],
                                   preferred_element_type=_f32)


_layer_call = pl.pallas_call(
    _layer_body,
    out_shape=(jax.ShapeDtypeStruct((N, H), _f32),
               jax.ShapeDtypeStruct((N, H), _f32)),
)


def _final_body(pp_ref, y_ref, dis_ref, m_ref, b_ref, wfc_ref, bfc_ref,
                out_ref):
    S = pp_ref[0:N, :] + pp_ref[NROW:NROW + N, :]
    h = jnp.maximum(dis_ref[...] * (S + y_ref[...]) + b_ref[...], 0.0)
    m = jnp.maximum(m_ref[...], h)
    logits = jnp.dot(m, wfc_ref[...], preferred_element_type=_f32)
    logits = logits + bfc_ref[...]
    lmax = jnp.max(logits, axis=1, keepdims=True)
    s = logits - lmax
    out_ref[...] = s - jnp.log(jnp.sum(jnp.exp(s), axis=1, keepdims=True))


_final_call = pl.pallas_call(
    _final_body,
    out_shape=jax.ShapeDtypeStruct((N, C), _f32),
)


def kernel(x, edge_index, W0, b0, W1, b1, W2, b2, W3, b3, W4, b4, W5, b5,
           Wfc, bfc):
    src = edge_index[0].astype(jnp.int32)
    dst = edge_index[1].astype(jnp.int32)
    pad = EPAD - E
    srcr = jnp.concatenate([src, jnp.zeros((pad,), jnp.int32)])
    dstr = jnp.concatenate([dst, jnp.full((pad,), N, jnp.int32)])
    srcr = srcr.reshape(EPAD // BLK, BLK)
    dstr = dstr.reshape(EPAD // BLK, BLK)

    dd = _deg_call(dstr)
    dis, y = _pre_call(dd, x, W0)

    m = jnp.zeros((N, H), _f32)
    Ws = [W1, W2, W3, W4, W5]
    bs = [b0.reshape(1, H), b1.reshape(1, H), b2.reshape(1, H),
          b3.reshape(1, H), b4.reshape(1, H)]
    for l in range(5):
        pp = _prop_call(y, srcr, dstr)
        y, m = _layer_call(pp, y, dis, m, bs[l], Ws[l])
    pp = _prop_call(y, srcr, dstr)
    return _final_call(pp, y, dis, m, b5.reshape(1, H), Wfc,
                       bfc.reshape(1, C))
